# baseline (speedup 1.0000x reference)
import functools

import jax
import jax.numpy as jnp
from jax import lax
from jax.experimental import pallas as pl
from jax.experimental.pallas import tpu as pltpu
from jax.experimental.pallas import tpu_sc as plsc

D = 128
NC = 2
NS = 16
NW = NC * NS
CHUNK = 128
CPP = 40
N_PAD = 10112
ROWS_PER_TILE = N_PAD // NS


def _make_edge_agg(c0: int, c1: int, cpp0: int, cpp1: int, n_pass: int,
                   row_len: int):
    assert c0 == cpp0 * n_pass and c1 == cpp1 * n_pass
    assert cpp0 % 2 == 0 and cpp1 % 2 == 0
    mesh = plsc.VectorSubcoreMesh(core_axis_name="c", subcore_axis_name="s")

    @functools.partial(
        pl.kernel,
        out_type=jax.ShapeDtypeStruct((NC, N_PAD, D), jnp.float32),
        mesh=mesh,
        scratch_types=[
            pltpu.VMEM((cpp0, CHUNK), jnp.int32),
            pltpu.VMEM((cpp0, CHUNK), jnp.int32),
            pltpu.VMEM((CHUNK, D), jnp.float32),
            pltpu.VMEM((CHUNK, D), jnp.float32),
            pltpu.VMEM_SHARED((N_PAD, D), jnp.float32),
            pltpu.SemaphoreType.DMA,
            pltpu.SemaphoreType.DMA,
        ],
    )
    def edge_agg(m_hbm, src_hbm, dst_hbm, zeros_hbm,
                 out_hbm, src_v, dst_v, rows0, rows1, agg_sh, sem0, sem1):
        cid = lax.axis_index("c")
        sid = lax.axis_index("s")
        wid = sid * NC + cid
        pltpu.sync_copy(zeros_hbm,
                        agg_sh.at[pl.ds(sid * ROWS_PER_TILE, ROWS_PER_TILE)])
        plsc.subcore_barrier()

        cpp = jnp.where(cid == 0, cpp0, cpp1)
        srow = src_hbm.at[wid]
        drow = dst_hbm.at[wid]

        for p in range(n_pass):
            slab0 = pl.multiple_of(p * cpp, 8)
            pltpu.sync_copy(srow.at[pl.ds(slab0, cpp0)], src_v)
            pltpu.sync_copy(drow.at[pl.ds(slab0, cpp0)], dst_v)
            pltpu.async_copy(m_hbm.at[src_v.at[0]], rows0, sem0)

            def body(k, carry):
                j0 = 2 * k
                j1 = j0 + 1
                pltpu.async_copy(m_hbm.at[src_v.at[j1]], rows1, sem1)
                pltpu.make_async_copy(
                    m_hbm.at[src_v.at[j0]], rows0, sem0).wait()
                pltpu.sync_copy(rows0, agg_sh.at[dst_v.at[j0]], add=True)

                @pl.when(j0 + 2 < cpp)
                def _():
                    pltpu.async_copy(m_hbm.at[src_v.at[j0 + 2]], rows0, sem0)

                pltpu.make_async_copy(
                    m_hbm.at[src_v.at[j1]], rows1, sem1).wait()
                pltpu.sync_copy(rows1, agg_sh.at[dst_v.at[j1]], add=True)
                return carry

            lax.fori_loop(0, cpp // 2, body, 0)
        plsc.subcore_barrier()
        pltpu.sync_copy(
            agg_sh.at[pl.ds(sid * ROWS_PER_TILE, ROWS_PER_TILE)],
            out_hbm.at[cid].at[pl.ds(sid * ROWS_PER_TILE, ROWS_PER_TILE)])

    return edge_agg


def _make_clicked_gather(b_pad: int):
    bpw = b_pad // NW
    mesh = plsc.VectorSubcoreMesh(core_axis_name="c", subcore_axis_name="s")

    @functools.partial(
        pl.kernel,
        out_type=jax.ShapeDtypeStruct((b_pad, D), jnp.float32),
        mesh=mesh,
        scratch_types=[
            pltpu.VMEM((bpw,), jnp.int32),
            pltpu.VMEM((bpw, D), jnp.float32),
            pltpu.SemaphoreType.DMA,
        ],
    )
    def clicked_gather(h_hbm, idx_hbm, out_hbm, idx_v, rows_v, sem):
        cid = lax.axis_index("c")
        sid = lax.axis_index("s")
        wid = sid * NC + cid
        base = wid * bpw
        pltpu.sync_copy(idx_hbm.at[pl.ds(base, bpw)], idx_v)
        pltpu.async_copy(h_hbm.at[idx_v], rows_v, sem).wait()
        pltpu.sync_copy(rows_v, out_hbm.at[pl.ds(base, bpw)])

    return clicked_gather


def _mm_body(x_ref, w_ref, o_ref):
    o_ref[...] = jnp.dot(x_ref[...], w_ref[...],
                         preferred_element_type=jnp.float32)


def _matmul(x, w, br=2000):
    n = x.shape[0]
    return pl.pallas_call(
        _mm_body,
        grid=(n // br,),
        in_specs=[
            pl.BlockSpec((br, D), lambda b: (b, 0)),
            pl.BlockSpec((D, D), lambda b: (0, 0)),
        ],
        out_specs=pl.BlockSpec((br, D), lambda b: (b, 0)),
        out_shape=jax.ShapeDtypeStruct((n, D), jnp.float32),
    )(x, w)


def _gru_math(p_ref, h_ref, wih_t_ref, whh_t_ref, bih_ref, bhh_ref):
    agg = p_ref[0] + p_ref[1]
    h = h_ref[...]
    gi = jnp.dot(agg, wih_t_ref[...],
                 preferred_element_type=jnp.float32) + bih_ref[...]
    gh = jnp.dot(h, whh_t_ref[...],
                 preferred_element_type=jnp.float32) + bhh_ref[...]
    r = jax.nn.sigmoid(gi[:, :D] + gh[:, :D])
    z = jax.nn.sigmoid(gi[:, D:2 * D] + gh[:, D:2 * D])
    n = jnp.tanh(gi[:, 2 * D:] + r * gh[:, 2 * D:])
    return (1.0 - z) * n + z * h


def _gru_body(p_ref, h_ref, wih_t_ref, whh_t_ref, bih_ref, bhh_ref, h_out_ref):
    h_out_ref[...] = _gru_math(p_ref, h_ref, wih_t_ref, whh_t_ref,
                               bih_ref, bhh_ref)


def _gru_mm_body(p_ref, h_ref, wih_t_ref, whh_t_ref, bih_ref, bhh_ref,
                 wn_ref, h_out_ref, m_out_ref):
    h_new = _gru_math(p_ref, h_ref, wih_t_ref, whh_t_ref, bih_ref, bhh_ref)
    h_out_ref[...] = h_new
    m_out_ref[...] = jnp.dot(h_new, wn_ref[...],
                             preferred_element_type=jnp.float32)


def _gru(parts, h, wih_t, whh_t, bih, bhh, w_next=None, br=2000):
    n = h.shape[0]
    grid = (n // br,)
    in_specs = [
        pl.BlockSpec((NC, br, D), lambda b: (0, b, 0)),
        pl.BlockSpec((br, D), lambda b: (b, 0)),
        pl.BlockSpec((D, 3 * D), lambda b: (0, 0)),
        pl.BlockSpec((D, 3 * D), lambda b: (0, 0)),
        pl.BlockSpec((1, 3 * D), lambda b: (0, 0)),
        pl.BlockSpec((1, 3 * D), lambda b: (0, 0)),
    ]
    if w_next is None:
        return pl.pallas_call(
            _gru_body,
            grid=grid,
            in_specs=in_specs,
            out_specs=pl.BlockSpec((br, D), lambda b: (b, 0)),
            out_shape=jax.ShapeDtypeStruct((n, D), jnp.float32),
        )(parts, h, wih_t, whh_t, bih, bhh)
    return pl.pallas_call(
        _gru_mm_body,
        grid=grid,
        in_specs=in_specs + [pl.BlockSpec((D, D), lambda b: (0, 0))],
        out_specs=[pl.BlockSpec((br, D), lambda b: (b, 0)),
                   pl.BlockSpec((br, D), lambda b: (b, 0))],
        out_shape=[jax.ShapeDtypeStruct((n, D), jnp.float32),
                   jax.ShapeDtypeStruct((n, D), jnp.float32)],
    )(parts, h, wih_t, whh_t, bih, bhh, w_next)


def kernel(x, weight, w_ih, w_hh, b_ih, b_hh, edge_index, mapping_idx):
    n_nodes, d = x.shape
    num_layers = weight.shape[0]
    e = edge_index.shape[1]
    batch, num_clicked = mapping_idx.shape

    c0, c1, n_pass = 112, 48, 2
    cpp0, cpp1 = c0 // n_pass, c1 // n_pass
    row_len = c0 + 1
    t_chunks = NS * (c0 + c1)
    e_pad = t_chunks * CHUNK
    src_flat = jnp.concatenate(
        [edge_index[0], jnp.zeros((e_pad - e,), jnp.int32)])
    dst_flat = jnp.concatenate(
        [edge_index[1], jnp.full((e_pad - e,), n_nodes, jnp.int32)])
    n0 = NS * c0 * CHUNK

    def layout(flat, fill):
        r0 = flat[:n0].reshape(NS, c0, CHUNK)
        r1 = flat[n0:].reshape(NS, c1, CHUNK)
        r0 = jnp.pad(r0, ((0, 0), (0, row_len - c0), (0, 0)),
                     constant_values=fill)
        r1 = jnp.pad(r1, ((0, 0), (0, row_len - c1), (0, 0)),
                     constant_values=fill)
        return jnp.stack([r0, r1], axis=1).reshape(NW, row_len, CHUNK)

    src = layout(src_flat, 0)
    dst = layout(dst_flat, n_nodes)
    zeros_tile = jnp.zeros((ROWS_PER_TILE, D), jnp.float32)

    wih_t = w_ih.T
    whh_t = w_hh.T
    bih = b_ih.reshape(1, 3 * D)
    bhh = b_hh.reshape(1, 3 * D)

    edge_agg = _make_edge_agg(c0, c1, cpp0, cpp1, n_pass, row_len)

    h = x
    m = _matmul(h, weight[0])
    for i in range(num_layers):
        parts = edge_agg(m, src, dst, zeros_tile)
        if i + 1 < num_layers:
            h, m = _gru(parts, h, wih_t, whh_t, bih, bhh, w_next=weight[i + 1])
        else:
            h = _gru(parts, h, wih_t, whh_t, bih, bhh)

    nb = batch * num_clicked
    bgran = 8 * NW
    b_pad = ((nb + bgran - 1) // bgran) * bgran
    idx_flat = jnp.concatenate(
        [mapping_idx.reshape(-1), jnp.zeros((b_pad - nb,), jnp.int32)])
    gathered = _make_clicked_gather(b_pad)(h, idx_flat)
    return gathered[:nb].reshape(batch, num_clicked, D)


# speedup vs baseline: 1.1861x; 1.1861x over previous
import functools

import jax
import jax.numpy as jnp
from jax import lax
from jax.experimental import pallas as pl
from jax.experimental.pallas import tpu as pltpu
from jax.experimental.pallas import tpu_sc as plsc

D = 128
NC = 2
NS = 16
NW = NC * NS
CHUNK = 128
CPP = 40
N_PAD = 10112
ROWS_PER_TILE = N_PAD // NS


def _make_edge_agg(c0: int, c1: int, cpp0: int, cpp1: int, n_pass: int,
                   row_len: int):
    assert c0 == cpp0 * n_pass and c1 == cpp1 * n_pass
    assert cpp0 % 2 == 0 and cpp1 % 2 == 0
    mesh = plsc.VectorSubcoreMesh(core_axis_name="c", subcore_axis_name="s")

    @functools.partial(
        pl.kernel,
        out_type=jax.ShapeDtypeStruct((NC, N_PAD, D), jnp.float32),
        mesh=mesh,
        scratch_types=[
            pltpu.VMEM((cpp0, CHUNK), jnp.int32),
            pltpu.VMEM((cpp0, CHUNK), jnp.int32),
            pltpu.VMEM((CHUNK, D), jnp.float32),
            pltpu.VMEM((CHUNK, D), jnp.float32),
            pltpu.VMEM_SHARED((N_PAD, D), jnp.float32),
            pltpu.SemaphoreType.DMA,
            pltpu.SemaphoreType.DMA,
        ],
    )
    def edge_agg(m_hbm, src_hbm, dst_hbm, zeros_hbm,
                 out_hbm, src_v, dst_v, rows0, rows1, agg_sh, sem0, sem1):
        cid = lax.axis_index("c")
        sid = lax.axis_index("s")
        wid = sid * NC + cid
        pltpu.sync_copy(zeros_hbm,
                        agg_sh.at[pl.ds(sid * ROWS_PER_TILE, ROWS_PER_TILE)])
        plsc.subcore_barrier()

        cpp = jnp.where(cid == 0, cpp0, cpp1)
        srow = src_hbm.at[wid]
        drow = dst_hbm.at[wid]

        for p in range(n_pass):
            slab0 = pl.multiple_of(p * cpp, 8)
            pltpu.sync_copy(srow.at[pl.ds(slab0, cpp0)], src_v)
            pltpu.sync_copy(drow.at[pl.ds(slab0, cpp0)], dst_v)
            pltpu.async_copy(m_hbm.at[src_v.at[0]], rows0, sem0)

            def body(k, carry):
                j0 = 2 * k
                j1 = j0 + 1
                pltpu.async_copy(m_hbm.at[src_v.at[j1]], rows1, sem1)
                pltpu.make_async_copy(
                    m_hbm.at[src_v.at[j0]], rows0, sem0).wait()
                pltpu.sync_copy(rows0, agg_sh.at[dst_v.at[j0]], add=True)

                @pl.when(j0 + 2 < cpp)
                def _():
                    pltpu.async_copy(m_hbm.at[src_v.at[j0 + 2]], rows0, sem0)

                pltpu.make_async_copy(
                    m_hbm.at[src_v.at[j1]], rows1, sem1).wait()
                pltpu.sync_copy(rows1, agg_sh.at[dst_v.at[j1]], add=True)
                return carry

            lax.fori_loop(0, cpp // 2, body, 0)
        plsc.subcore_barrier()
        pltpu.sync_copy(
            agg_sh.at[pl.ds(sid * ROWS_PER_TILE, ROWS_PER_TILE)],
            out_hbm.at[cid].at[pl.ds(sid * ROWS_PER_TILE, ROWS_PER_TILE)])

    return edge_agg


def _make_clicked_gather(b_pad: int):
    bpw = b_pad // NW
    mesh = plsc.VectorSubcoreMesh(core_axis_name="c", subcore_axis_name="s")

    @functools.partial(
        pl.kernel,
        out_type=jax.ShapeDtypeStruct((b_pad, D), jnp.float32),
        mesh=mesh,
        scratch_types=[
            pltpu.VMEM((bpw,), jnp.int32),
            pltpu.VMEM((bpw, D), jnp.float32),
            pltpu.SemaphoreType.DMA,
        ],
    )
    def clicked_gather(h_hbm, idx_hbm, out_hbm, idx_v, rows_v, sem):
        cid = lax.axis_index("c")
        sid = lax.axis_index("s")
        wid = sid * NC + cid
        base = wid * bpw
        pltpu.sync_copy(idx_hbm.at[pl.ds(base, bpw)], idx_v)
        pltpu.async_copy(h_hbm.at[idx_v], rows_v, sem).wait()
        pltpu.sync_copy(rows_v, out_hbm.at[pl.ds(base, bpw)])

    return clicked_gather


def _mm_body(x_ref, w_ref, o_ref):
    o_ref[...] = jnp.dot(x_ref[...], w_ref[...],
                         preferred_element_type=jnp.float32)


def _matmul(x, w, br=2000):
    n = x.shape[0]
    return pl.pallas_call(
        _mm_body,
        grid=(n // br,),
        in_specs=[
            pl.BlockSpec((br, D), lambda b: (b, 0)),
            pl.BlockSpec((D, D), lambda b: (0, 0)),
        ],
        out_specs=pl.BlockSpec((br, D), lambda b: (b, 0)),
        out_shape=jax.ShapeDtypeStruct((n, D), jnp.float32),
    )(x, w)


def _gru_math(p_ref, h_ref, wih_t_ref, whh_t_ref, bih_ref, bhh_ref):
    agg = p_ref[0] + p_ref[1]
    h = h_ref[...]
    gi = jnp.dot(agg, wih_t_ref[...],
                 preferred_element_type=jnp.float32) + bih_ref[...]
    gh = jnp.dot(h, whh_t_ref[...],
                 preferred_element_type=jnp.float32) + bhh_ref[...]
    r = jax.nn.sigmoid(gi[:, :D] + gh[:, :D])
    z = jax.nn.sigmoid(gi[:, D:2 * D] + gh[:, D:2 * D])
    n = jnp.tanh(gi[:, 2 * D:] + r * gh[:, 2 * D:])
    return (1.0 - z) * n + z * h


def _gru_body(p_ref, h_ref, wih_t_ref, whh_t_ref, bih_ref, bhh_ref, h_out_ref):
    h_out_ref[...] = _gru_math(p_ref, h_ref, wih_t_ref, whh_t_ref,
                               bih_ref, bhh_ref)


def _gru_mm_body(p_ref, h_ref, wih_t_ref, whh_t_ref, bih_ref, bhh_ref,
                 wn_ref, h_out_ref, m_out_ref):
    h_new = _gru_math(p_ref, h_ref, wih_t_ref, whh_t_ref, bih_ref, bhh_ref)
    h_out_ref[...] = h_new
    m_out_ref[...] = jnp.dot(h_new, wn_ref[...],
                             preferred_element_type=jnp.float32)


def _gru(parts, h, wih_t, whh_t, bih, bhh, w_next=None, br=2000):
    n = h.shape[0]
    grid = (n // br,)
    in_specs = [
        pl.BlockSpec((NC, br, D), lambda b: (0, b, 0)),
        pl.BlockSpec((br, D), lambda b: (b, 0)),
        pl.BlockSpec((D, 3 * D), lambda b: (0, 0)),
        pl.BlockSpec((D, 3 * D), lambda b: (0, 0)),
        pl.BlockSpec((1, 3 * D), lambda b: (0, 0)),
        pl.BlockSpec((1, 3 * D), lambda b: (0, 0)),
    ]
    if w_next is None:
        return pl.pallas_call(
            _gru_body,
            grid=grid,
            in_specs=in_specs,
            out_specs=pl.BlockSpec((br, D), lambda b: (b, 0)),
            out_shape=jax.ShapeDtypeStruct((n, D), jnp.float32),
        )(parts, h, wih_t, whh_t, bih, bhh)
    return pl.pallas_call(
        _gru_mm_body,
        grid=grid,
        in_specs=in_specs + [pl.BlockSpec((D, D), lambda b: (0, 0))],
        out_specs=[pl.BlockSpec((br, D), lambda b: (b, 0)),
                   pl.BlockSpec((br, D), lambda b: (b, 0))],
        out_shape=[jax.ShapeDtypeStruct((n, D), jnp.float32),
                   jax.ShapeDtypeStruct((n, D), jnp.float32)],
    )(parts, h, wih_t, whh_t, bih, bhh, w_next)


def kernel(x, weight, w_ih, w_hh, b_ih, b_hh, edge_index, mapping_idx):
    n_nodes, d = x.shape
    num_layers = weight.shape[0]
    e = edge_index.shape[1]
    batch, num_clicked = mapping_idx.shape

    c0, c1, n_pass = 96, 64, 2
    cpp0, cpp1 = c0 // n_pass, c1 // n_pass
    row_len = c0 + 1
    t_chunks = NS * (c0 + c1)
    e_pad = t_chunks * CHUNK
    src_flat = jnp.concatenate(
        [edge_index[0], jnp.zeros((e_pad - e,), jnp.int32)])
    dst_flat = jnp.concatenate(
        [edge_index[1], jnp.full((e_pad - e,), n_nodes, jnp.int32)])
    n0 = NS * c0 * CHUNK

    def layout(flat, fill):
        r0 = flat[:n0].reshape(NS, c0, CHUNK)
        r1 = flat[n0:].reshape(NS, c1, CHUNK)
        r0 = jnp.pad(r0, ((0, 0), (0, row_len - c0), (0, 0)),
                     constant_values=fill)
        r1 = jnp.pad(r1, ((0, 0), (0, row_len - c1), (0, 0)),
                     constant_values=fill)
        return jnp.stack([r0, r1], axis=1).reshape(NW, row_len, CHUNK)

    src = layout(src_flat, 0)
    dst = layout(dst_flat, n_nodes)
    zeros_tile = jnp.zeros((ROWS_PER_TILE, D), jnp.float32)

    wih_t = w_ih.T
    whh_t = w_hh.T
    bih = b_ih.reshape(1, 3 * D)
    bhh = b_hh.reshape(1, 3 * D)

    edge_agg = _make_edge_agg(c0, c1, cpp0, cpp1, n_pass, row_len)

    h = x
    m = _matmul(h, weight[0])
    for i in range(num_layers):
        parts = edge_agg(m, src, dst, zeros_tile)
        if i + 1 < num_layers:
            h, m = _gru(parts, h, wih_t, whh_t, bih, bhh, w_next=weight[i + 1])
        else:
            h = _gru(parts, h, wih_t, whh_t, bih, bhh)

    nb = batch * num_clicked
    bgran = 8 * NW
    b_pad = ((nb + bgran - 1) // bgran) * bgran
    idx_flat = jnp.concatenate(
        [mapping_idx.reshape(-1), jnp.zeros((b_pad - nb,), jnp.int32)])
    gathered = _make_clicked_gather(b_pad)(h, idx_flat)
    return gathered[:nb].reshape(batch, num_clicked, D)
